# baseline (speedup 1.0000x reference)
import jax, jax.numpy as jnp
import numpy as np
from jax.experimental import pallas as pl
from jax.experimental.pallas import tpu as pltpu

_R = 58 * 58
_M = 64
_BUF = _M + _R + 68
_NPTS = 4 * 56 * 56
_NELEM = float(_NPTS * 64)
_K = 512


def _kof1(d, r):
    return {(-1, 0): 3, (0, 0): 1, (0, 1): 2, (1, 1): 0}.get((d, r))


def _kof2(d, py, cy):
    if d == 0:
        k = cy - 2 * py + 1
        return k if 0 <= k <= 3 else None
    if d == 1:
        return 0 if (py == 0 and cy == 3) else None
    return 3 if (py == 1 and cy == 0) else None


def _build_sels():
    s1 = np.zeros((9, 2, 2, 4, 4), np.float32)
    s2 = np.zeros((9, 2, 2, 4, 4, 4, 4), np.float32)
    for si, (dy, dx) in enumerate((dy, dx) for dy in (-1, 0, 1) for dx in (-1, 0, 1)):
        for a in range(2):
            for b in range(2):
                ky, kx = _kof1(dy, a), _kof1(dx, b)
                if ky is not None and kx is not None:
                    s1[si, a, b, ky, kx] = 1.0
                for cy in range(4):
                    for cx in range(4):
                        ky2, kx2 = _kof2(dy, a, cy), _kof2(dx, b, cx)
                        if ky2 is not None and kx2 is not None:
                            s2[si, a, b, cy, cx, ky2, kx2] = 1.0
    return s1, s2


_SEL1, _SEL2 = _build_sels()


def _conv2d(x, w, b, stride=1, pad=0):
    out = jax.lax.conv_general_dilated(x, w, (stride, stride), [(pad, pad), (pad, pad)],
                                       dimension_numbers=('NCHW', 'OIHW', 'NCHW'))
    return out + b[None, :, None, None]


def _group_norm(x, g, b, groups=32, eps=1e-5):
    N, C, H, W = x.shape
    xr = x.reshape(N, groups, C // groups, H, W)
    m = xr.mean(axis=(2, 3, 4), keepdims=True)
    v = xr.var(axis=(2, 3, 4), keepdims=True)
    xr = (xr - m) / jnp.sqrt(v + eps)
    x = xr.reshape(N, C, H, W)
    return x * g[None, :, None, None] + b[None, :, None, None]


def _res_block(x, p, pre):
    idn = x
    out = jax.nn.relu(_group_norm(_conv2d(x, p[pre + '_conv1_w'], p[pre + '_conv1_b'], 1, 1),
                                  p[pre + '_gn1_g'], p[pre + '_gn1_b']))
    out = _group_norm(_conv2d(out, p[pre + '_conv2_w'], p[pre + '_conv2_b'], 1, 0),
                      p[pre + '_gn2_g'], p[pre + '_gn2_b'])
    return jax.nn.relu(out + idn)


def _fused_kernel(z_ref, zsq_ref, mask_ref, g_mat_ref, cbt_ref, csq_ref, cb_ref,
                  wpv_ref, bpv_ref, wdc1_ref, bdc1_ref,
                  w0c1_ref, b0c1_ref, g0a_ref, b0a_ref,
                  w0c2_ref, b0c2_ref, g0b_ref, b0b_ref,
                  w1c1_ref, b1c1_ref, g1a_ref, b1a_ref,
                  w1c2_ref, b1c2_ref, g1b_ref, b1b_ref,
                  wct1_ref, bct1_ref, wct2_ref, bct2_ref,
                  out_ref, loss_ref, perp_ref,
                  buf_a, buf_b, buf_c, loss_acc, hist_acc):
    step = pl.program_id(0)
    bf16 = jnp.bfloat16

    @pl.when(step == 0)
    def _init():
        buf_a[...] = jnp.zeros_like(buf_a)
        buf_b[...] = jnp.zeros_like(buf_b)
        buf_c[...] = jnp.zeros_like(buf_c)
        loss_acc[...] = jnp.zeros_like(loss_acc)
        hist_acc[...] = jnp.zeros_like(hist_acc)

    mask = mask_ref[...]
    m64 = mask[:, :64]
    m1 = mask[:, :1]

    z = z_ref[0]
    s = jnp.dot(z, cbt_ref[...], preferred_element_type=jnp.float32)
    d = (zsq_ref[0, :, :1] + csq_ref[...]) - 2.0 * s
    dmin = jnp.min(d, axis=1, keepdims=True)
    lane = jax.lax.broadcasted_iota(jnp.int32, d.shape, 1)
    idx = jnp.min(jnp.where(d == dmin, lane, _K), axis=1, keepdims=True)
    enc = jnp.where(lane == idx, 1.0, 0.0).astype(jnp.float32)
    q = jnp.dot(enc, cb_ref[...], preferred_element_type=jnp.float32)
    qst = z + (q - z)
    diff = (q - z) * m64
    loss_acc[...] += jnp.sum(diff * diff).reshape(1, 1)
    hist_acc[...] += jnp.sum(enc * m1, axis=0, keepdims=True)
    qm = qst * m64

    def conv3x3(buf, wt_ref):
        acc = None
        for dy in range(3):
            for dx in range(3):
                o = _M + (dy - 1) * 58 + (dx - 1)
                t = jnp.dot(buf[o:o + _R, :], wt_ref[dy, dx],
                            preferred_element_type=jnp.float32)
                acc = t if acc is None else acc + t
        return acc

    def gn(t, g_row, b_row):
        s1 = jnp.sum(t, axis=0, keepdims=True)
        s2 = jnp.sum(t * t, axis=0, keepdims=True)
        gs1 = jnp.dot(s1, g_mat_ref[...], preferred_element_type=jnp.float32,
                      precision=jax.lax.Precision.HIGHEST)
        gs2 = jnp.dot(s2, g_mat_ref[...], preferred_element_type=jnp.float32,
                      precision=jax.lax.Precision.HIGHEST)
        m = gs1 / 12544.0
        v = gs2 / 12544.0 - m * m
        inv = jax.lax.rsqrt(v + 1e-5)
        return (t - m) * inv * g_row[...] + b_row[...]

    h = (jnp.dot(qm.astype(bf16), wpv_ref[...], preferred_element_type=jnp.float32)
         + bpv_ref[...]) * mask
    buf_a[_M:_M + _R, :] = h.astype(bf16)
    h1 = (conv3x3(buf_a, wdc1_ref) + bdc1_ref[...]) * mask

    def res_block(hin, wc1, bc1, ga, ba, wc2, bc2, gb, bb):
        buf_b[_M:_M + _R, :] = hin.astype(bf16)
        t = (conv3x3(buf_b, wc1) + bc1[...]) * mask
        t = jax.nn.relu(gn(t, ga, ba)) * mask
        u = jnp.dot(t.astype(bf16), wc2[...], preferred_element_type=jnp.float32) + bc2[...]
        u = gn(u * mask, gb, bb)
        return jax.nn.relu(u + hin) * mask

    h2 = res_block(h1, w0c1_ref, b0c1_ref, g0a_ref, b0a_ref,
                   w0c2_ref, b0c2_ref, g0b_ref, b0b_ref)
    h3 = res_block(h2, w1c1_ref, b1c1_ref, g1a_ref, b1a_ref,
                   w1c2_ref, b1c2_ref, g1b_ref, b1b_ref)

    buf_a[_M:_M + _R, :] = h3.astype(bf16)
    shifts = [(dy, dx) for dy in (-1, 0, 1) for dx in (-1, 0, 1)]
    acc = None
    for si, (dy, dx) in enumerate(shifts):
        o = _M + dy * 58 + dx
        t = jnp.dot(buf_a[o:o + _R, :], wct1_ref[si],
                    preferred_element_type=jnp.float32)
        acc = t if acc is None else acc + t
    u = jax.nn.relu(acc + bct1_ref[...]) * m1

    buf_c[_M:_M + _R, :] = u.astype(bf16)
    acc = None
    for si, (dy, dx) in enumerate(shifts):
        o = _M + dy * 58 + dx
        t = jnp.dot(buf_c[o:o + _R, :], wct2_ref[si],
                    preferred_element_type=jnp.float32)
        acc = t if acc is None else acc + t
    out_ref[0] = jnp.tanh(acc + bct2_ref[...])

    @pl.when(step == 3)
    def _fin():
        loss_ref[...] = loss_acc[...] / _NELEM
        avg = hist_acc[...] / float(_NPTS)
        ent = jnp.sum(avg * jnp.log(avg + 1e-10)).reshape(1, 1)
        perp_ref[...] = jnp.exp(-ent)


def _full(shape):
    nd = len(shape)
    return pl.BlockSpec(shape, lambda i: (0,) * nd)


def _fused_pipeline(z_nhwc, p):
    bf = jnp.bfloat16
    flat = z_nhwc.reshape(-1, 64)
    zsq = jnp.sum(flat ** 2, axis=1, keepdims=True)
    zsqp = jnp.pad(zsq.reshape(4, 56, 56), ((0, 0), (1, 1), (1, 1))).reshape(4, _R, 1)
    zsqp = jnp.broadcast_to(zsqp, (4, _R, 8))
    zp = jnp.pad(z_nhwc, ((0, 0), (1, 1), (1, 1), (0, 0))).reshape(4, _R, 64)

    cbt = p['codebook'].T
    csq = jnp.sum(p['codebook'] ** 2, axis=1)[None, :]

    mrow = ((jnp.arange(58) >= 1) & (jnp.arange(58) <= 56)).astype(jnp.float32)
    m58 = (mrow[:, None] * mrow[None, :]).reshape(_R, 1) * jnp.ones((1, 128), jnp.float32)
    g_mat = jnp.kron(jnp.eye(32, dtype=jnp.float32), jnp.ones((4, 4), jnp.float32))

    def t33(w):
        return jnp.transpose(w, (2, 3, 1, 0)).astype(bf)
    def t11(w):
        return w[:, :, 0, 0].T.astype(bf)
    def row(b):
        return b[None, :]

    wct1 = jnp.einsum('ioyx,sabyx->siabo', p['dec_ct1_w'],
                      _SEL1).reshape(9, 128, 256).astype(bf)
    bct1 = jnp.tile(p['dec_ct1_b'], 4)[None, :]

    wct2 = jnp.einsum('ioyx,sabcdyx->sabicdo', p['dec_ct2_w'],
                      _SEL2).reshape(9, 256, 48).astype(bf)
    bct2 = jnp.tile(p['dec_ct2_b'], 16)[None, :]

    args = [zp, zsqp, m58, g_mat, cbt, csq, p['codebook'],
            t11(p['post_vq_w']), row(p['post_vq_b']),
            t33(p['dec_conv1_w']), row(p['dec_conv1_b'])]
    for pre in ('dec_res0', 'dec_res1'):
        args += [t33(p[pre + '_conv1_w']), row(p[pre + '_conv1_b']),
                 row(p[pre + '_gn1_g']), row(p[pre + '_gn1_b']),
                 t11(p[pre + '_conv2_w']), row(p[pre + '_conv2_b']),
                 row(p[pre + '_gn2_g']), row(p[pre + '_gn2_b'])]
    args += [wct1, bct1, wct2, bct2]

    planes, loss, perp = pl.pallas_call(
        _fused_kernel,
        grid=(4,),
        in_specs=[pl.BlockSpec((1, _R, 64), lambda i: (i, 0, 0)),
                  pl.BlockSpec((1, _R, 8), lambda i: (i, 0, 0))]
                 + [_full(a.shape) for a in args[2:]],
        out_specs=[pl.BlockSpec((1, _R, 48), lambda i: (i, 0, 0)),
                   pl.BlockSpec((1, 1), lambda i: (0, 0)),
                   pl.BlockSpec((1, 1), lambda i: (0, 0))],
        out_shape=[jax.ShapeDtypeStruct((4, _R, 48), jnp.float32),
                   jax.ShapeDtypeStruct((1, 1), jnp.float32),
                   jax.ShapeDtypeStruct((1, 1), jnp.float32)],
        scratch_shapes=[pltpu.VMEM((_BUF, 128), jnp.bfloat16),
                        pltpu.VMEM((_BUF, 128), jnp.bfloat16),
                        pltpu.VMEM((_BUF, 256), jnp.bfloat16),
                        pltpu.VMEM((1, 1), jnp.float32),
                        pltpu.VMEM((1, _K), jnp.float32)],
    )(*args)

    pl2 = planes.reshape(4, 58, 58, 4, 4, 3)[:, 1:57, 1:57, :, :, :]
    recon = jnp.transpose(pl2, (0, 1, 3, 2, 4, 5)).reshape(4, 224, 224, 3)
    recon = jnp.transpose(recon, (0, 3, 1, 2))
    vq_loss = loss[0, 0]
    commit_loss = loss[0, 0] * 1.0
    return recon, vq_loss, commit_loss, perp[0, 0]


def kernel(x, params):
    p = params
    z = jax.nn.relu(_conv2d(x, p['enc_conv_in_w'], p['enc_conv_in_b'], 2, 1))
    z = jax.nn.relu(_conv2d(z, p['enc_conv1_w'], p['enc_conv1_b'], 2, 1))
    z = _conv2d(z, p['enc_conv2_w'], p['enc_conv2_b'], 1, 1)
    z = _res_block(z, p, 'enc_res0')
    z = _res_block(z, p, 'enc_res1')
    z = _conv2d(z, p['pre_vq_w'], p['pre_vq_b'], 1, 0)
    z_nhwc = jnp.transpose(z, (0, 2, 3, 1))
    recon, vq_loss, commit_loss, perp = _fused_pipeline(z_nhwc, p)
    return recon, vq_loss, commit_loss, perp
